# baseline (speedup 1.0000x reference)
import functools

import jax
import jax.numpy as jnp
from jax import lax
from jax.experimental import pallas as pl
from jax.experimental.pallas import tpu as pltpu

_B = 16
_D = 768
_RES = 512
_K = 64
_FG = 32
_INV_T = float(1.0 / 0.3)
_ROWS = 2048
_LANES = 128
_GROUPS = 256


def _stats_kernel(w_ref, selp_ref, fhfw_ref):
    big = jnp.int32(1 << 30)
    neg = jnp.float32(-jnp.inf)
    giota = (lax.broadcasted_iota(jnp.int32, (2, 128), 0) * 128
             + lax.broadcasted_iota(jnp.int32, (2, 128), 1))

    def init_g(g, carry):
        gval, gidx = carry
        blk = w_ref[0, pl.ds(g * 8, 8), :]
        flat = ((g * 8 + lax.broadcasted_iota(jnp.int32, (8, 128), 0)) * 128
                + lax.broadcasted_iota(jnp.int32, (8, 128), 1))
        m = jnp.max(blk)
        mi = jnp.min(jnp.where(blk == m, flat, big))
        gval = jnp.where(giota == g, m, gval)
        gidx = jnp.where(giota == g, mi, gidx)
        return gval, gidx

    gval, gidx = lax.fori_loop(
        0, _GROUPS, init_g,
        (jnp.full((2, 128), neg, jnp.float32),
         jnp.full((2, 128), big, jnp.int32)))

    maxw = jnp.max(gval)
    inv_t = jnp.float32(_INV_T)

    def denom_step(i, acc):
        slab = w_ref[0, pl.ds(i * 256, 256), :]
        return acc + jnp.sum(jnp.exp((slab - maxw) * inv_t))

    denom = lax.fori_loop(0, 8, denom_step, jnp.float32(0.0))

    kiota = lax.broadcasted_iota(jnp.int32, (1, 64), 1)

    def extract(k, carry):
        gval, gidx, topv, topidx = carry
        m = jnp.max(gval)
        cand = jnp.min(jnp.where(gval == m, gidx, big))
        g = cand // 1024
        row0 = g * 8
        blk = w_ref[0, pl.ds(row0, 8), :]
        flat = ((row0 + lax.broadcasted_iota(jnp.int32, (8, 128), 0)) * 128
                + lax.broadcasted_iota(jnp.int32, (8, 128), 1))
        blk = jnp.where(flat == cand, neg, blk)
        w_ref[0, pl.ds(row0, 8), :] = blk
        nm = jnp.max(blk)
        ni = jnp.min(jnp.where(blk == nm, flat, big))
        gval = jnp.where(giota == g, nm, gval)
        gidx = jnp.where(giota == g, ni, gidx)
        topv = jnp.where(kiota == k, m, topv)
        topidx = jnp.where(kiota == k, cand, topidx)
        return gval, gidx, topv, topidx

    _, _, topv, topidx = lax.fori_loop(
        0, _K, extract,
        (gval, gidx, jnp.zeros((1, 64), jnp.float32),
         jnp.zeros((1, 64), jnp.int32)))

    selp_ref[0] = jnp.exp((topv - maxw) * inv_t) / denom
    h = topidx // _RES
    wp = topidx - h * _RES
    fh = jnp.minimum(h // 16, _FG - 4)
    fw = jnp.minimum(wp // 16, _FG - 4)
    fhfw_ref[0] = jnp.concatenate([fh, fw], axis=0)


def _gather_kernel(fhfw_ref, selp_ref, feat_ref, out_ref):
    def body(k, c):
        fh = fhfw_ref[0, 0, k]
        fw = fhfw_ref[0, 1, k]
        p = selp_ref[0, 0, k]
        base = fh * _FG + fw
        for oh in range(4):
            r = base + _FG * oh
            w0 = jnp.minimum((r // 8) * 8, _FG * _FG - 16)
            w0 = pl.multiple_of(w0, 8)
            blk16 = feat_ref[0, pl.ds(w0, 16), :]
            rows4 = pltpu.roll(blk16, 16 - (r - w0), axis=0)[0:4, :]
            out_ref[0, k, pl.ds(4 * oh, 4), :] = rows4 * p
        return c

    lax.fori_loop(0, _K, body, 0)


@jax.jit
def kernel(features, weights):
    w3 = weights.reshape(_B, _ROWS, _LANES)
    selp, fhfw = pl.pallas_call(
        _stats_kernel,
        grid=(_B,),
        in_specs=[pl.BlockSpec((1, _ROWS, _LANES), lambda b: (b, 0, 0))],
        out_specs=[pl.BlockSpec((1, 1, 64), lambda b: (b, 0, 0)),
                   pl.BlockSpec((1, 2, 64), lambda b: (b, 0, 0))],
        out_shape=[jax.ShapeDtypeStruct((_B, 1, 64), jnp.float32),
                   jax.ShapeDtypeStruct((_B, 2, 64), jnp.int32)],
    )(w3)

    out = pl.pallas_call(
        _gather_kernel,
        grid=(_B,),
        in_specs=[
            pl.BlockSpec((1, 2, 64), lambda b: (b, 0, 0),
                         memory_space=pltpu.SMEM),
            pl.BlockSpec((1, 1, 64), lambda b: (b, 0, 0),
                         memory_space=pltpu.SMEM),
            pl.BlockSpec((1, _FG * _FG, _D), lambda b: (b, 0, 0)),
        ],
        out_specs=pl.BlockSpec((1, _K, 16, _D), lambda b: (b, 0, 0, 0)),
        out_shape=jax.ShapeDtypeStruct((_B, _K, 16, _D), jnp.float32),
    )(fhfw, selp, features)
    return out
